# Initial kernel scaffold; baseline (speedup 1.0000x reference)
import jax
import jax.numpy as jnp
from jax.experimental import pallas as pl


def kernel(x):
    raise NotImplementedError("write your pallas kernel here")


# speedup vs baseline: 24.8576x; 24.8576x over previous
import functools

import jax
import jax.numpy as jnp
from jax import lax
from jax.experimental import pallas as pl
from jax.experimental.pallas import tpu as pltpu
from jax.experimental.pallas import tpu_sc as plsc

N_BINS = 32
FEATS = 128
N_ROWS = 262144
NUM_WORKERS = 32
ROWS_PER_W = N_ROWS // NUM_WORKERS
CHUNK = 128
NCHUNK = ROWS_PER_W // CHUNK
LANES = 16
HIST = N_BINS * FEATS


def _sc_histogram(x2d):
    mesh = plsc.VectorSubcoreMesh(core_axis_name="c", subcore_axis_name="s")

    @functools.partial(
        pl.kernel,
        mesh=mesh,
        out_type=jax.ShapeDtypeStruct((NUM_WORKERS, HIST), jnp.float32),
        compiler_params=pltpu.CompilerParams(needs_layout_passes=False),
        scratch_types=[
            pltpu.VMEM((2, CHUNK, FEATS), jnp.float32),
            pltpu.VMEM((HIST,), jnp.float32),
            pltpu.SemaphoreType.DMA,
            pltpu.SemaphoreType.DMA,
        ],
    )
    def sc_hist(x_hbm, parts_hbm, buf, hist, sem0, sem1):
        cid = lax.axis_index("c")
        sid = lax.axis_index("s")
        wid = sid * 2 + cid
        base = wid * ROWS_PER_W

        zeros16 = jnp.zeros((LANES,), jnp.float32)

        def zero_body(i, _):
            hist[pl.ds(i * LANES, LANES)] = zeros16
            return 0

        lax.fori_loop(0, HIST // LANES, zero_body, 0)

        sems = (sem0, sem1)

        def copy(k, b):
            return pltpu.make_async_copy(
                x_hbm.at[pl.ds(base + k * CHUNK, CHUNK), :],
                buf.at[b],
                sems[b],
            )

        ones16 = jnp.full((LANES,), 1.0, jnp.float32)
        iota16 = lax.iota(jnp.int32, LANES)
        fvecs = [iota16 + (j * LANES) for j in range(FEATS // LANES)]

        def compute(b):
            def row_body(r, _):
                for j in range(FEATS // LANES):
                    v = buf[b, r, pl.ds(j * LANES, LANES)]
                    m = 32.0 / (1.0 + jnp.exp(-v))
                    i = jnp.minimum(m.astype(jnp.int32), 31)
                    flat = (i << 7) + fvecs[j]
                    plsc.addupdate_scatter(hist, [flat], ones16)
                return 0

            lax.fori_loop(0, CHUNK, row_body, 0)

        copy(0, 0).start()

        def outer(g, _):
            for b in range(2):
                k = 2 * g + b
                nxt = k + 1

                @pl.when(nxt < NCHUNK)
                def _():
                    copy(nxt, 1 - b).start()

                copy(k, b).wait()
                compute(b)
            return 0

        lax.fori_loop(0, NCHUNK // 2, outer, 0)

        pltpu.sync_copy(hist, parts_hbm.at[wid])

    return sc_hist(x2d)


def _tc_finish(parts):
    def body(parts_ref, out_ref):
        a = parts_ref[...]
        rows = lax.broadcasted_iota(jnp.int32, (NUM_WORKERS * N_BINS, N_BINS), 0)
        cols = lax.broadcasted_iota(jnp.int32, (NUM_WORKERS * N_BINS, N_BINS), 1)
        sel = jnp.where((rows % N_BINS) == cols, 1.0, 0.0).astype(jnp.float32)
        out = lax.dot_general(
            a, sel, (((0,), (0,)), ((), ())),
            preferred_element_type=jnp.float32,
        )
        out_ref[...] = out * (1.0 / N_ROWS)

    return pl.pallas_call(
        body,
        out_shape=jax.ShapeDtypeStruct((FEATS, N_BINS), jnp.float32),
    )(parts)


def kernel(x):
    x2d = x.reshape(N_ROWS, FEATS)
    parts = _sc_histogram(x2d)
    return _tc_finish(parts.reshape(NUM_WORKERS * N_BINS, FEATS))
